# baseline (speedup 1.0000x reference)
import functools

import jax
import jax.numpy as jnp
from jax import lax
from jax.experimental import pallas as pl
from jax.experimental.pallas import tpu as pltpu
from jax.experimental.pallas import tpu_sc as plsc


def _make_gather(B, V, D):
    info = plsc.get_sparse_core_info()
    nc, ns, L = info.num_cores, info.num_subcores, info.num_lanes
    nw = nc * ns
    assert B % nw == 0 and D % L == 0 and V % 8 == 0
    b_per_w = B // nw
    C = 128
    nch = b_per_w // C
    qs = D // L
    mesh = plsc.VectorSubcoreMesh(core_axis_name="c", subcore_axis_name="s")

    @functools.partial(
        pl.kernel,
        mesh=mesh,
        out_type=jax.ShapeDtypeStruct((B, D), jnp.float32),
        scratch_types=[
            pltpu.VMEM((b_per_w,), jnp.int32),
            pltpu.VMEM((nch, C), jnp.int32),
            pltpu.VMEM((C, 8, D), jnp.float32),
            pltpu.VMEM((b_per_w, D), jnp.float32),
            pltpu.SemaphoreType.DMA,
        ],
    )
    def gather_k(idx_hbm, table3_hbm, out_hbm, idx_v, tidx_v, tiles_v, rows_v, sem):
        wid = lax.axis_index("s") * nc + lax.axis_index("c")
        base = wid * b_per_w
        pltpu.sync_copy(idx_hbm.at[pl.ds(base, b_per_w)], idx_v)
        pltpu.sync_copy(rows_v, out_hbm.at[pl.ds(base, b_per_w)])

    return gather_k


def kernel(u_idx, user_table):
    (B,) = u_idx.shape
    V, D = user_table.shape
    return _make_gather(B, V, D)(u_idx.astype(jnp.int32), user_table)


# speedup vs baseline: 1.0703x; 1.0703x over previous
import functools

import jax
import jax.numpy as jnp
from jax import lax
from jax.experimental import pallas as pl
from jax.experimental.pallas import tpu as pltpu
from jax.experimental.pallas import tpu_sc as plsc


def _make_gather(B, V, D):
    info = plsc.get_sparse_core_info()
    nc, ns, L = info.num_cores, info.num_subcores, info.num_lanes
    nw = nc * ns
    assert B % nw == 0 and D % 8 == 0
    b_per_w = B // nw
    A = D // 8
    NB = 4
    mesh = plsc.VectorSubcoreMesh(core_axis_name="c", subcore_axis_name="s")

    @functools.partial(
        pl.kernel,
        mesh=mesh,
        out_type=jax.ShapeDtypeStruct((A, 8, B), jnp.float32),
        scratch_types=[
            pltpu.VMEM((b_per_w,), jnp.int32),
            pltpu.VMEM((A, 8, 128), jnp.float32),
            pltpu.VMEM((A, 8, 128), jnp.float32),
            pltpu.VMEM((A, 8, 128), jnp.float32),
            pltpu.VMEM((A, 8, 128), jnp.float32),
            pltpu.VMEM((A, 8, b_per_w), jnp.float32),
            pltpu.VMEM_SHARED((ns, A, 8, b_per_w), jnp.float32),
            pltpu.SemaphoreType.DMA,
            pltpu.SemaphoreType.DMA,
        ],
    )
    def gather_k(
        idx_hbm, table3_hbm, out3_hbm, idx_v, w0, w1, w2, w3, xfer_v, shared, sem, esem
    ):
        wins = [w0, w1, w2, w3]
        cid = lax.axis_index("c")
        sid = lax.axis_index("s")
        wid = sid * nc + cid
        base = wid * b_per_w
        pltpu.sync_copy(idx_hbm.at[pl.ds(base, b_per_w)], idx_v)

        def group(g, carry):
            vec = idx_v[pl.ds(g * L, L)]
            lvec = vec & 127
            for h in range(L // NB):
                for j in range(NB):
                    r = vec[h * NB + j]
                    rt = pl.multiple_of((r >> 7) * 128, 128)
                    pltpu.async_copy(
                        table3_hbm.at[:, :, pl.ds(rt, 128)], wins[j], sem
                    )
                for j in range(NB):
                    pltpu.make_async_copy(
                        table3_hbm.at[:, :, pl.ds(0, 128)], wins[j], sem
                    ).wait()
                for j in range(NB):
                    b = g * L + h * NB + j
                    l = lvec[h * NB + j]
                    pltpu.async_copy(
                        wins[j].at[:, :, pl.ds(l, 1)],
                        shared.at[sid, :, :, pl.ds(b, 1)],
                        esem,
                    )
                for j in range(NB):
                    pltpu.make_async_copy(
                        wins[j].at[:, :, pl.ds(0, 1)],
                        shared.at[sid, :, :, pl.ds(0, 1)],
                        esem,
                    ).wait()
            return carry

        lax.fori_loop(0, b_per_w // L, group, 0)
        pltpu.sync_copy(shared.at[sid], xfer_v)
        pltpu.sync_copy(xfer_v, out3_hbm.at[:, :, pl.ds(base, b_per_w)])

    return gather_k


def kernel(u_idx, user_table):
    (B,) = u_idx.shape
    V, D = user_table.shape
    table3 = user_table.T.reshape(D // 8, 8, V)
    out3 = _make_gather(B, V, D)(u_idx.astype(jnp.int32), table3)
    return out3.reshape(D, B).T


# speedup vs baseline: 1.6846x; 1.5740x over previous
import functools

import jax
import jax.numpy as jnp
from jax import lax
from jax.experimental import pallas as pl
from jax.experimental.pallas import tpu as pltpu
from jax.experimental.pallas import tpu_sc as plsc


def _make_gather(B, V, D):
    info = plsc.get_sparse_core_info()
    nc, ns, L = info.num_cores, info.num_subcores, info.num_lanes
    nw = nc * ns
    assert B % nw == 0 and D % 8 == 0
    b_per_w = B // nw
    A = D // 8
    NB = 8
    rt_max = (V // 128) * 128 - 128
    mesh = plsc.VectorSubcoreMesh(core_axis_name="c", subcore_axis_name="s")

    @functools.partial(
        pl.kernel,
        mesh=mesh,
        out_type=jax.ShapeDtypeStruct((A, 8, B), jnp.float32),
        scratch_types=[
            pltpu.VMEM((b_per_w + L,), jnp.int32),
            [pltpu.VMEM((A, 8, 128), jnp.float32) for _ in range(NB)],
            pltpu.VMEM_SHARED((ns, A, 8, b_per_w), jnp.float32),
            pltpu.SemaphoreType.DMA,
            pltpu.SemaphoreType.DMA,
        ],
    )
    def gather_k(
        idx_hbm, table3_hbm, out3_hbm, idx_v, wins, shared, sem, esem
    ):
        cid = lax.axis_index("c")
        sid = lax.axis_index("s")
        wid = sid * nc + cid
        base = wid * b_per_w
        idx_v[pl.ds(b_per_w, L)] = lax.iota(jnp.int32, L) * 0
        pltpu.sync_copy(idx_hbm.at[pl.ds(base, b_per_w)], idx_v.at[pl.ds(0, b_per_w)])

        def fire_fetch(r, k):
            rt = jnp.minimum((r >> 7) * 128, rt_max)
            rt = pl.multiple_of(rt, 128)
            pltpu.async_copy(table3_hbm.at[:, :, pl.ds(rt, 128)], wins[k], sem)

        vec0 = idx_v[pl.ds(0, L)]
        for j in range(NB):
            fire_fetch(vec0[j], j)

        def group(g, carry):
            vec = idx_v[pl.ds(g * L, L)]
            vecn = idx_v[pl.ds(g * L + L, L)]
            lvec = vec & 127
            for j in range(L):
                p = g * L + j
                k = j % NB
                pltpu.make_async_copy(
                    table3_hbm.at[:, :, pl.ds(0, 128)], wins[k], sem
                ).wait()
                l = lvec[j]
                pltpu.async_copy(
                    wins[k].at[:, :, pl.ds(l, 1)],
                    shared.at[sid, :, :, pl.ds(p, 1)],
                    esem,
                )
                pltpu.make_async_copy(
                    wins[k].at[:, :, pl.ds(0, 1)],
                    shared.at[sid, :, :, pl.ds(0, 1)],
                    esem,
                ).wait()
                r_next = vec[j + NB] if j + NB < L else vecn[j + NB - L]
                fire_fetch(r_next, k)
            return carry

        lax.fori_loop(0, b_per_w // L, group, 0)
        for k in range(NB):
            pltpu.make_async_copy(
                table3_hbm.at[:, :, pl.ds(0, 128)], wins[k], sem
            ).wait()
        pltpu.sync_copy(shared.at[sid], out3_hbm.at[:, :, pl.ds(base, b_per_w)])

    return gather_k


def kernel(u_idx, user_table):
    (B,) = u_idx.shape
    V, D = user_table.shape
    table3 = user_table.T.reshape(D // 8, 8, V)
    out3 = _make_gather(B, V, D)(u_idx.astype(jnp.int32), table3)
    return out3.reshape(D, B).T
